# baseline (speedup 1.0000x reference)
import functools

import jax
import jax.numpy as jnp
from jax import lax
from jax.experimental import pallas as pl
from jax.experimental.pallas import tpu as pltpu
from jax.experimental.pallas import tpu_sc as plsc

NC = 2
NS = 16
NW = NC * NS
L = 16
CH = 2048


def _make_sc_kernel(n, e, h, zpad):
    e_chunks = -(-e // (NW * CH))
    h_chunks = -(-h // (NW * CH))

    mesh = plsc.VectorSubcoreMesh(
        core_axis_name="c", subcore_axis_name="s", num_cores=NC,
        num_subcores=NS)

    @functools.partial(
        pl.kernel,
        out_type=jax.ShapeDtypeStruct((NC, zpad), jnp.float32),
        mesh=mesh,
        scratch_types=[
            pltpu.VMEM((n,), jnp.float32),
            pltpu.VMEM((3 * CH,), jnp.int32),
            pltpu.VMEM((CH,), jnp.int32),
            pltpu.VMEM((CH,), jnp.int32),
            pltpu.VMEM((CH,), jnp.int32),
            pltpu.VMEM((CH,), jnp.float32),
            pltpu.VMEM((CH,), jnp.float32),
            pltpu.VMEM((CH,), jnp.float32),
            pltpu.MemorySpace.VMEM_SHARED((zpad,), jnp.float32),
        ],
        compiler_params=pltpu.CompilerParams(needs_layout_passes=False),
    )
    def sc_kernel(x_hbm, ef_hbm, hf_hbm, z_hbm, out_hbm,
                  x_v, hbuf, ia, ib, ic, va, vb, vc, acc):
        c = lax.axis_index("c")
        s = lax.axis_index("s")
        wid = c * NS + s
        iota = lax.iota(jnp.int32, L)
        iota3 = iota * 3

        @pl.when(s == 0)
        def _zero():
            pltpu.sync_copy(z_hbm, acc)

        pltpu.sync_copy(x_hbm, x_v)
        plsc.subcore_barrier()

        def edge_chunk(k, carry):
            g = k * NW + wid
            s0 = g * CH
            m0 = jnp.minimum(s0, e - CH)
            delta = s0 - m0
            pltpu.sync_copy(ef_hbm.at[pl.ds(m0, CH)], ia)
            pltpu.sync_copy(ef_hbm.at[pl.ds(e + m0, CH)], ib)

            def vec(j, carry2):
                sl = pl.ds(j * L, L)
                msk = (j * L + iota) >= delta
                si = ia[sl]
                di = ib[sl]
                xs = plsc.load_gather(x_v, [si])
                xd = plsc.load_gather(x_v, [di])
                va[sl] = jnp.where(msk, -0.1 * (xd - xs), 0.0)
                return carry2
            lax.fori_loop(0, CH // L, vec, carry)
            pltpu.sync_copy(va, acc.at[ib], add=True)
            return carry
        lax.fori_loop(0, e_chunks, edge_chunk, 0)

        def hyper_chunk(k, carry):
            g = k * NW + wid
            s0 = g * CH
            m0 = jnp.minimum(s0, h - CH)
            delta = s0 - m0
            pltpu.sync_copy(hf_hbm.at[pl.ds(3 * m0, 3 * CH)], hbuf)

            def vec(j, carry2):
                sl = pl.ds(j * L, L)
                msk = (j * L + iota) >= delta
                b = 3 * L * j
                i1 = plsc.load_gather(hbuf, [iota3 + b])
                i2 = plsc.load_gather(hbuf, [iota3 + (b + 1)])
                i3 = plsc.load_gather(hbuf, [iota3 + (b + 2)])
                ia[sl] = i1
                ib[sl] = i2
                ic[sl] = i3
                x1 = plsc.load_gather(x_v, [i1])
                x2 = plsc.load_gather(x_v, [i2])
                x3 = plsc.load_gather(x_v, [i3])
                p = x2 * x3
                va[sl] = jnp.where(msk, 0.9 * (p - x1 * x1), 0.0)
                vb[sl] = jnp.where(msk, 0.9 * (p - x2 * x2), 0.0)
                vc[sl] = jnp.where(msk, 0.9 * (p - x3 * x3), 0.0)
                return carry2
            lax.fori_loop(0, CH // L, vec, carry)
            pltpu.sync_copy(va, acc.at[ia], add=True)
            pltpu.sync_copy(vb, acc.at[ib], add=True)
            pltpu.sync_copy(vc, acc.at[ic], add=True)
            return carry
        lax.fori_loop(0, h_chunks, hyper_chunk, 0)

        plsc.subcore_barrier()

        @pl.when(s == 0)
        def _out():
            pltpu.sync_copy(acc, out_hbm.at[c])

    return sc_kernel


def _sum2_body(parts_ref, o_ref):
    o_ref[...] = parts_ref[0, :] + parts_ref[1, :]


def kernel(t, x, edge_index, hyperedges):
    del t
    n = x.shape[0]
    e = edge_index.shape[1]
    h = hyperedges.shape[0]
    zpad = -(-n // (NS * 128)) * (NS * 128)

    ef = edge_index.astype(jnp.int32).reshape(-1)
    hf = hyperedges.astype(jnp.int32).reshape(-1)
    z = jnp.zeros((zpad,), jnp.float32)

    parts = _make_sc_kernel(n, e, h, zpad)(x, ef, hf, z)

    summed = pl.pallas_call(
        _sum2_body,
        out_shape=jax.ShapeDtypeStruct((zpad,), jnp.float32),
    )(parts)
    return summed[:n]


# speedup vs baseline: 1.7361x; 1.7361x over previous
import functools

import jax
import jax.numpy as jnp
from jax import lax
from jax.experimental import pallas as pl
from jax.experimental.pallas import tpu as pltpu
from jax.experimental.pallas import tpu_sc as plsc

NC = 2
NS = 16
NW = NC * NS
L = 16
CH = 2048

EB = 25600
HB = 5120


def _edge_split_body(e_ref, s_ref, d_ref):
    s_ref[...] = e_ref[0]
    d_ref[...] = e_ref[1]


def _hyper_split_body(h_ref, a_ref, b_ref, c_ref):
    a_ref[...] = h_ref[:, 0]
    b_ref[...] = h_ref[:, 1]
    c_ref[...] = h_ref[:, 2]


def _make_sc_kernel(n, e, h, zpad):
    e_chunks = -(-e // (NW * CH))
    h_chunks = -(-h // (NW * CH))

    mesh = plsc.VectorSubcoreMesh(
        core_axis_name="c", subcore_axis_name="s", num_cores=NC,
        num_subcores=NS)

    @functools.partial(
        pl.kernel,
        out_type=jax.ShapeDtypeStruct((NC, zpad), jnp.float32),
        mesh=mesh,
        scratch_types=[
            pltpu.VMEM((n,), jnp.float32),
            pltpu.VMEM((CH,), jnp.int32),
            pltpu.VMEM((CH,), jnp.int32),
            pltpu.VMEM((CH,), jnp.int32),
            pltpu.VMEM((CH,), jnp.float32),
            pltpu.VMEM((CH,), jnp.float32),
            pltpu.VMEM((CH,), jnp.float32),
            pltpu.MemorySpace.VMEM_SHARED((zpad,), jnp.float32),
        ],
        compiler_params=pltpu.CompilerParams(needs_layout_passes=False),
    )
    def sc_kernel(x_hbm, src_hbm, dst_hbm, h1_hbm, h2_hbm, h3_hbm, z_hbm,
                  out_hbm, x_v, ia, ib, ic, va, vb, vc, acc):
        c = lax.axis_index("c")
        s = lax.axis_index("s")
        wid = c * NS + s
        iota = lax.iota(jnp.int32, L)

        @pl.when(s == 0)
        def _zero():
            pltpu.sync_copy(z_hbm, acc)

        pltpu.sync_copy(x_hbm, x_v)
        plsc.subcore_barrier()

        def edge_chunk(k, carry):
            g = k * NW + wid
            s0 = g * CH
            m0 = jnp.minimum(s0, e - CH)
            delta = s0 - m0
            pltpu.sync_copy(src_hbm.at[pl.ds(m0, CH)], ia)
            pltpu.sync_copy(dst_hbm.at[pl.ds(m0, CH)], ib)

            def vec(j, carry2):
                sl = pl.ds(j * L, L)
                msk = (j * L + iota) >= delta
                si = jnp.where(msk, ia[sl], 0)
                di = jnp.where(msk, ib[sl], 0)
                ib[sl] = di
                xs = plsc.load_gather(x_v, [si])
                xd = plsc.load_gather(x_v, [di])
                va[sl] = jnp.where(msk, -0.1 * (xd - xs), 0.0)
                return carry2
            lax.fori_loop(0, CH // L, vec, carry)
            pltpu.sync_copy(va, acc.at[ib], add=True)
            return carry
        lax.fori_loop(0, e_chunks, edge_chunk, 0)

        def hyper_chunk(k, carry):
            g = k * NW + wid
            s0 = g * CH
            m0 = jnp.minimum(s0, h - CH)
            delta = s0 - m0
            pltpu.sync_copy(h1_hbm.at[pl.ds(m0, CH)], ia)
            pltpu.sync_copy(h2_hbm.at[pl.ds(m0, CH)], ib)
            pltpu.sync_copy(h3_hbm.at[pl.ds(m0, CH)], ic)

            def vec(j, carry2):
                sl = pl.ds(j * L, L)
                msk = (j * L + iota) >= delta
                i1 = jnp.where(msk, ia[sl], 0)
                i2 = jnp.where(msk, ib[sl], 0)
                i3 = jnp.where(msk, ic[sl], 0)
                ia[sl] = i1
                ib[sl] = i2
                ic[sl] = i3
                x1 = plsc.load_gather(x_v, [i1])
                x2 = plsc.load_gather(x_v, [i2])
                x3 = plsc.load_gather(x_v, [i3])
                p = x2 * x3
                va[sl] = jnp.where(msk, 0.9 * (p - x1 * x1), 0.0)
                vb[sl] = jnp.where(msk, 0.9 * (p - x2 * x2), 0.0)
                vc[sl] = jnp.where(msk, 0.9 * (p - x3 * x3), 0.0)
                return carry2
            lax.fori_loop(0, CH // L, vec, carry)
            pltpu.sync_copy(va, acc.at[ia], add=True)
            pltpu.sync_copy(vb, acc.at[ib], add=True)
            pltpu.sync_copy(vc, acc.at[ic], add=True)
            return carry
        lax.fori_loop(0, h_chunks, hyper_chunk, 0)

        plsc.subcore_barrier()

        @pl.when(s == 0)
        def _out():
            pltpu.sync_copy(acc, out_hbm.at[c])

    return sc_kernel


def _sum2_body(parts_ref, o_ref):
    o_ref[...] = parts_ref[0, :] + parts_ref[1, :]


def kernel(t, x, edge_index, hyperedges):
    del t
    n = x.shape[0]
    e = edge_index.shape[1]
    h = hyperedges.shape[0]
    zpad = -(-n // (NS * 128)) * (NS * 128)

    ei = edge_index.astype(jnp.int32)
    he = hyperedges.astype(jnp.int32)

    eg = -(-e // EB)
    epad = eg * EB
    src, dst = pl.pallas_call(
        _edge_split_body,
        grid=(eg,),
        in_specs=[pl.BlockSpec((2, EB), lambda i: (0, i))],
        out_specs=[pl.BlockSpec((EB,), lambda i: (i,)),
                   pl.BlockSpec((EB,), lambda i: (i,))],
        out_shape=[jax.ShapeDtypeStruct((epad,), jnp.int32),
                   jax.ShapeDtypeStruct((epad,), jnp.int32)],
    )(ei)

    hg = -(-h // HB)
    hpad = hg * HB
    h1, h2, h3 = pl.pallas_call(
        _hyper_split_body,
        grid=(hg,),
        in_specs=[pl.BlockSpec((HB, 3), lambda i: (i, 0))],
        out_specs=[pl.BlockSpec((HB,), lambda i: (i,)),
                   pl.BlockSpec((HB,), lambda i: (i,)),
                   pl.BlockSpec((HB,), lambda i: (i,))],
        out_shape=[jax.ShapeDtypeStruct((hpad,), jnp.int32),
                   jax.ShapeDtypeStruct((hpad,), jnp.int32),
                   jax.ShapeDtypeStruct((hpad,), jnp.int32)],
    )(he)

    z = jnp.zeros((zpad,), jnp.float32)

    parts = _make_sc_kernel(n, e, h, zpad)(x, src, dst, h1, h2, h3, z)

    summed = pl.pallas_call(
        _sum2_body,
        out_shape=jax.ShapeDtypeStruct((zpad,), jnp.float32),
    )(parts)
    return summed[:n]


# speedup vs baseline: 3.9009x; 2.2469x over previous
import functools

import jax
import jax.numpy as jnp
from jax import lax
from jax.experimental import pallas as pl
from jax.experimental.pallas import tpu as pltpu
from jax.experimental.pallas import tpu_sc as plsc

NC = 2
NS = 16
NW = NC * NS
L = 16
CH = 2048


def _make_sc_kernel(n, epad, hpad, zpad):
    e_chunks = epad // (NW * CH)
    h_chunks = hpad // (NW * CH)

    mesh = plsc.VectorSubcoreMesh(
        core_axis_name="c", subcore_axis_name="s", num_cores=NC,
        num_subcores=NS)

    @functools.partial(
        pl.kernel,
        out_type=jax.ShapeDtypeStruct((NC, zpad), jnp.float32),
        mesh=mesh,
        scratch_types=[
            pltpu.VMEM((CH,), jnp.int32),
            pltpu.VMEM((CH,), jnp.int32),
            pltpu.VMEM((CH,), jnp.int32),
            pltpu.VMEM((CH,), jnp.float32),
            pltpu.VMEM((CH,), jnp.float32),
            pltpu.VMEM((CH,), jnp.float32),
            pltpu.MemorySpace.VMEM_SHARED((zpad,), jnp.float32),
            pltpu.MemorySpace.VMEM_SHARED((zpad,), jnp.float32),
        ],
        compiler_params=pltpu.CompilerParams(needs_layout_passes=False),
    )
    def sc_kernel(x_hbm, src_hbm, dst_hbm, h1_hbm, h2_hbm, h3_hbm, z_hbm,
                  out_hbm, ia, ib, ic, ga, gb, gc, x_sh, acc):
        c = lax.axis_index("c")
        s = lax.axis_index("s")
        wid = c * NS + s

        @pl.when(s == 0)
        def _zero():
            pltpu.sync_copy(z_hbm, acc)

        @pl.when(s == 1)
        def _stage():
            pltpu.sync_copy(x_hbm, x_sh)

        plsc.subcore_barrier()

        def edge_chunk(k, carry):
            s0 = (k * NW + wid) * CH
            pltpu.sync_copy(src_hbm.at[pl.ds(s0, CH)], ia)
            pltpu.sync_copy(dst_hbm.at[pl.ds(s0, CH)], ib)
            pltpu.sync_copy(x_sh.at[ia], ga)
            pltpu.sync_copy(x_sh.at[ib], gb)

            def vec(j, carry2):
                sl = pl.ds(j * L, L)
                ga[sl] = -0.1 * (gb[sl] - ga[sl])
                return carry2
            lax.fori_loop(0, CH // L, vec, carry)
            pltpu.sync_copy(ga, acc.at[ib], add=True)
            return carry
        lax.fori_loop(0, e_chunks, edge_chunk, 0)

        def hyper_chunk(k, carry):
            s0 = (k * NW + wid) * CH
            pltpu.sync_copy(h1_hbm.at[pl.ds(s0, CH)], ia)
            pltpu.sync_copy(h2_hbm.at[pl.ds(s0, CH)], ib)
            pltpu.sync_copy(h3_hbm.at[pl.ds(s0, CH)], ic)
            pltpu.sync_copy(x_sh.at[ia], ga)
            pltpu.sync_copy(x_sh.at[ib], gb)
            pltpu.sync_copy(x_sh.at[ic], gc)

            def vec(j, carry2):
                sl = pl.ds(j * L, L)
                a = ga[sl]
                b = gb[sl]
                cc = gc[sl]
                p = b * cc
                ga[sl] = 0.9 * (p - a * a)
                gb[sl] = 0.9 * (p - b * b)
                gc[sl] = 0.9 * (p - cc * cc)
                return carry2
            lax.fori_loop(0, CH // L, vec, carry)
            pltpu.sync_copy(ga, acc.at[ia], add=True)
            pltpu.sync_copy(gb, acc.at[ib], add=True)
            pltpu.sync_copy(gc, acc.at[ic], add=True)
            return carry
        lax.fori_loop(0, h_chunks, hyper_chunk, 0)

        plsc.subcore_barrier()

        @pl.when(s == 0)
        def _out():
            pltpu.sync_copy(acc, out_hbm.at[c])

    return sc_kernel


def _sum2_body(parts_ref, o_ref):
    o_ref[...] = parts_ref[0, :] + parts_ref[1, :]


def kernel(t, x, edge_index, hyperedges):
    del t
    n = x.shape[0]
    e = edge_index.shape[1]
    h = hyperedges.shape[0]
    blk = NW * CH
    zpad = -(-n // (NS * 128)) * (NS * 128)
    epad = -(-e // blk) * blk
    hpad = -(-h // blk) * blk

    ei = edge_index.astype(jnp.int32)
    he = hyperedges.astype(jnp.int32)

    src = jnp.pad(ei[0], (0, epad - e))
    dst = jnp.pad(ei[1], (0, epad - e))
    h1 = jnp.pad(he[:, 0], (0, hpad - h))
    h2 = jnp.pad(he[:, 1], (0, hpad - h))
    h3 = jnp.pad(he[:, 2], (0, hpad - h))

    z = jnp.zeros((zpad,), jnp.float32)
    xp = jnp.pad(x, (0, zpad - n))

    parts = _make_sc_kernel(n, epad, hpad, zpad)(xp, src, dst, h1, h2, h3, z)

    summed = pl.pallas_call(
        _sum2_body,
        out_shape=jax.ShapeDtypeStruct((zpad,), jnp.float32),
    )(parts)
    return summed[:n]


# speedup vs baseline: 5.0975x; 1.3067x over previous
import functools

import jax
import jax.numpy as jnp
from jax import lax
from jax.experimental import pallas as pl
from jax.experimental.pallas import tpu as pltpu
from jax.experimental.pallas import tpu_sc as plsc

NC = 2
NS = 16
NW = NC * NS
L = 16
CH = 2048


def _make_sc_kernel(n, epad, hpad, zpad):
    e_chunks = epad // (NW * CH)
    h_chunks = hpad // (NW * CH)

    mesh = plsc.VectorSubcoreMesh(
        core_axis_name="c", subcore_axis_name="s", num_cores=NC,
        num_subcores=NS)

    @functools.partial(
        pl.kernel,
        out_type=jax.ShapeDtypeStruct((NC, zpad), jnp.float32),
        mesh=mesh,
        scratch_types=[
            pltpu.VMEM((CH,), jnp.int32),
            pltpu.VMEM((CH,), jnp.int32),
            pltpu.VMEM((CH,), jnp.int32),
            pltpu.VMEM((CH,), jnp.float32),
            pltpu.VMEM((CH,), jnp.float32),
            pltpu.VMEM((CH,), jnp.float32),
            pltpu.VMEM((zpad,), jnp.float32),
            pltpu.MemorySpace.VMEM_SHARED((zpad,), jnp.float32),
        ],
        compiler_params=pltpu.CompilerParams(needs_layout_passes=False),
    )
    def sc_kernel(x_hbm, src_hbm, dst_hbm, h1_hbm, h2_hbm, h3_hbm, z_hbm,
                  out_hbm, ia, ib, ic, ga, gb, gc, x_sh, acc):
        c = lax.axis_index("c")
        s = lax.axis_index("s")
        wid = c * NS + s

        @pl.when(s == 0)
        def _zero():
            pltpu.sync_copy(z_hbm, acc)

        pltpu.sync_copy(x_hbm, x_sh)

        plsc.subcore_barrier()

        def edge_chunk(k, carry):
            s0 = (k * NW + wid) * CH
            pltpu.sync_copy(src_hbm.at[pl.ds(s0, CH)], ia)
            pltpu.sync_copy(dst_hbm.at[pl.ds(s0, CH)], ib)

            def vec(j, carry2):
                sl = pl.ds(j * L, L)
                xs = plsc.load_gather(x_sh, [ia[sl]])
                xd = plsc.load_gather(x_sh, [ib[sl]])
                ga[sl] = -0.1 * (xd - xs)
                return carry2
            lax.fori_loop(0, CH // L, vec, carry)
            pltpu.sync_copy(ga, acc.at[ib], add=True)
            return carry
        lax.fori_loop(0, e_chunks, edge_chunk, 0)

        def hyper_chunk(k, carry):
            s0 = (k * NW + wid) * CH
            pltpu.sync_copy(h1_hbm.at[pl.ds(s0, CH)], ia)
            pltpu.sync_copy(h2_hbm.at[pl.ds(s0, CH)], ib)
            pltpu.sync_copy(h3_hbm.at[pl.ds(s0, CH)], ic)

            def vec(j, carry2):
                sl = pl.ds(j * L, L)
                a = plsc.load_gather(x_sh, [ia[sl]])
                b = plsc.load_gather(x_sh, [ib[sl]])
                cc = plsc.load_gather(x_sh, [ic[sl]])
                p = b * cc
                ga[sl] = 0.9 * (p - a * a)
                gb[sl] = 0.9 * (p - b * b)
                gc[sl] = 0.9 * (p - cc * cc)
                return carry2
            lax.fori_loop(0, CH // L, vec, carry)
            pltpu.sync_copy(ga, acc.at[ia], add=True)
            pltpu.sync_copy(gb, acc.at[ib], add=True)
            pltpu.sync_copy(gc, acc.at[ic], add=True)
            return carry
        lax.fori_loop(0, h_chunks, hyper_chunk, 0)

        plsc.subcore_barrier()

        @pl.when(s == 0)
        def _out():
            pltpu.sync_copy(acc, out_hbm.at[c])

    return sc_kernel


def _sum2_body(parts_ref, o_ref):
    o_ref[...] = parts_ref[0, :] + parts_ref[1, :]


def kernel(t, x, edge_index, hyperedges):
    del t
    n = x.shape[0]
    e = edge_index.shape[1]
    h = hyperedges.shape[0]
    blk = NW * CH
    zpad = -(-n // (NS * 128)) * (NS * 128)
    epad = -(-e // blk) * blk
    hpad = -(-h // blk) * blk

    ei = edge_index.astype(jnp.int32)
    he = hyperedges.astype(jnp.int32)

    src = jnp.pad(ei[0], (0, epad - e))
    dst = jnp.pad(ei[1], (0, epad - e))
    h1 = jnp.pad(he[:, 0], (0, hpad - h))
    h2 = jnp.pad(he[:, 1], (0, hpad - h))
    h3 = jnp.pad(he[:, 2], (0, hpad - h))

    z = jnp.zeros((zpad,), jnp.float32)
    xp = jnp.pad(x, (0, zpad - n))

    parts = _make_sc_kernel(n, epad, hpad, zpad)(xp, src, dst, h1, h2, h3, z)

    summed = pl.pallas_call(
        _sum2_body,
        out_shape=jax.ShapeDtypeStruct((zpad,), jnp.float32),
    )(parts)
    return summed[:n]
